# baseline (speedup 1.0000x reference)
import jax
import jax.numpy as jnp
from jax import lax
from jax.experimental import pallas as pl
from jax.experimental.pallas import tpu as pltpu
from jax.experimental.pallas import tpu_sc as plsc

B = 16384
D = 32
N_CORES = 2
N_SUBCORES = 16
NW = N_CORES * N_SUBCORES
BPW = B // NW
CHUNK = 128
NCHUNK = BPW // CHUNK
LANES = 16


def _body(user_hbm, arm_hbm, ut_hbm, at_hbm, out_hbm,
          idx_u, idx_a, rows_u, rows_a, out_v, sem):
    wid = lax.axis_index("s") * N_CORES + lax.axis_index("c")
    rbase = wid * NCHUNK

    pltpu.sync_copy(user_hbm.at[pl.ds(rbase, NCHUNK)], idx_u)
    pltpu.sync_copy(arm_hbm.at[pl.ds(rbase, NCHUNK)], idx_a)

    handles = []
    for j in range(NCHUNK):
        handles.append(pltpu.async_copy(
            ut_hbm.at[idx_u.at[j]], rows_u.at[pl.ds(j * CHUNK, CHUNK)], sem))
        handles.append(pltpu.async_copy(
            at_hbm.at[idx_a.at[j]], rows_a.at[pl.ds(j * CHUNK, CHUNK)], sem))
    for h in handles:
        h.wait()

    def group(g, carry):
        rows = g * LANES + lax.iota(jnp.int32, LANES)
        acc = jnp.zeros((LANES,), jnp.float32)
        for d in range(D):
            col = jnp.full((LANES,), d, jnp.int32)
            cu = plsc.load_gather(rows_u, [rows, col])
            ca = plsc.load_gather(rows_a, [rows, col])
            acc = acc + cu * ca
        out_v[pl.ds(g * LANES, LANES)] = acc
        return carry

    lax.fori_loop(0, BPW // LANES, group, jnp.int32(0))

    pltpu.sync_copy(out_v, out_hbm.at[pl.ds(wid * BPW, BPW)])


@jax.jit
def kernel(user, arm, user_table, arm_table):
    user2d = user.astype(jnp.int32).reshape(B // CHUNK, CHUNK)
    arm2d = arm.astype(jnp.int32).reshape(B // CHUNK, CHUNK)
    mesh = plsc.VectorSubcoreMesh(core_axis_name="c", subcore_axis_name="s",
                                  num_cores=N_CORES, num_subcores=N_SUBCORES)
    f = pl.kernel(
        _body,
        out_type=jax.ShapeDtypeStruct((B,), jnp.float32),
        mesh=mesh,
        scratch_types=[
            pltpu.VMEM((NCHUNK, CHUNK), jnp.int32),
            pltpu.VMEM((NCHUNK, CHUNK), jnp.int32),
            pltpu.VMEM((BPW, D), jnp.float32),
            pltpu.VMEM((BPW, D), jnp.float32),
            pltpu.VMEM((BPW,), jnp.float32),
            pltpu.SemaphoreType.DMA,
        ],
        compiler_params=pltpu.CompilerParams(needs_layout_passes=False,
                                             use_tc_tiling_on_sc=False),
    )
    return f(user2d, arm2d, user_table, arm_table)
